# Initial kernel scaffold; baseline (speedup 1.0000x reference)
import jax
import jax.numpy as jnp
from jax.experimental import pallas as pl


def kernel(token_ids, weight):
    raise NotImplementedError("write your pallas kernel here")


# speedup vs baseline: 2.9634x; 2.9634x over previous
import functools

import jax
import jax.numpy as jnp
from jax import lax
from jax.experimental import pallas as pl
from jax.experimental.pallas import tpu as pltpu
from jax.experimental.pallas import tpu_sc as plsc

NUM_TOKENS = 4096 * 50
DIM = 128
NC, NS = 2, 16
NW = NC * NS
CHUNK = 128
CHUNKS_PER_W = NUM_TOKENS // (NW * CHUNK)


def _make_kernel():
    mesh = plsc.VectorSubcoreMesh(core_axis_name="c", subcore_axis_name="s")

    @functools.partial(
        pl.kernel,
        out_type=jax.ShapeDtypeStruct((NUM_TOKENS, DIM), jnp.float32),
        mesh=mesh,
        scratch_types=[
            pltpu.VMEM((CHUNKS_PER_W, CHUNK), jnp.int32),
            pltpu.VMEM((CHUNK, DIM), jnp.float32),
            pltpu.SemaphoreType.DMA,
        ],
    )
    def gather_kernel(idx_hbm, table_hbm, out_hbm, idx_v, rows_v, sem):
        wid = lax.axis_index("s") * NC + lax.axis_index("c")
        pltpu.sync_copy(idx_hbm.at[wid], idx_v)
        base = wid * (CHUNKS_PER_W * CHUNK)

        def step(j, carry):
            pltpu.async_copy(table_hbm.at[idx_v.at[j]], rows_v, sem).wait()
            pltpu.sync_copy(rows_v, out_hbm.at[pl.ds(base + j * CHUNK, CHUNK)])
            return carry

        lax.fori_loop(0, CHUNKS_PER_W, step, 0)

    return gather_kernel


_gather = _make_kernel()


def kernel(token_ids, weight):
    idx = token_ids.astype(jnp.int32).reshape(NW, CHUNKS_PER_W, CHUNK)
    out = _gather(idx, weight)
    return out.reshape(token_ids.shape + (DIM,))


# speedup vs baseline: 3.3137x; 1.1182x over previous
import functools

import jax
import jax.numpy as jnp
from jax import lax
from jax.experimental import pallas as pl
from jax.experimental.pallas import tpu as pltpu
from jax.experimental.pallas import tpu_sc as plsc

NUM_TOKENS = 4096 * 50
DIM = 128
NC, NS = 2, 16
NW = NC * NS
CHUNK = 128
CHUNKS_PER_W = NUM_TOKENS // (NW * CHUNK)
RING = 5


def _make_kernel():
    mesh = plsc.VectorSubcoreMesh(core_axis_name="c", subcore_axis_name="s")

    @functools.partial(
        pl.kernel,
        out_type=jax.ShapeDtypeStruct((NUM_TOKENS, DIM), jnp.float32),
        mesh=mesh,
        scratch_types=[
            pltpu.VMEM((CHUNKS_PER_W, CHUNK), jnp.int32),
            pltpu.VMEM((RING, CHUNK, DIM), jnp.float32),
            pltpu.SemaphoreType.DMA((RING,)),
            pltpu.SemaphoreType.DMA((RING,)),
        ],
    )
    def gather_kernel(idx_hbm, table_hbm, out_hbm, idx_v, rows_v, sem_g, sem_o):
        wid = lax.axis_index("s") * NC + lax.axis_index("c")
        pltpu.sync_copy(idx_hbm.at[wid], idx_v)
        base = wid * (CHUNKS_PER_W * CHUNK)

        def gather(j, b):
            return pltpu.make_async_copy(
                table_hbm.at[idx_v.at[j]], rows_v.at[b], sem_g.at[b])

        def outcopy(j, b):
            return pltpu.make_async_copy(
                rows_v.at[b], out_hbm.at[pl.ds(base + j * CHUNK, CHUNK)],
                sem_o.at[b])

        for b in range(RING):
            gather(b, b).start()

        @pl.loop(0, CHUNKS_PER_W - RING, step=RING)
        def _outer(t):
            for b in range(RING):
                gather(t + b, b).wait()
                outcopy(t + b, b).start()
            for b in range(RING):
                outcopy(t + b, b).wait()
                gather(t + RING + b, b).start()

        t = CHUNKS_PER_W - RING
        for b in range(RING):
            gather(t + b, b).wait()
            outcopy(t + b, b).start()
        for b in range(RING):
            outcopy(t + b, b).wait()

    return gather_kernel


_gather = _make_kernel()


def kernel(token_ids, weight):
    idx = token_ids.astype(jnp.int32).reshape(NW, CHUNKS_PER_W, CHUNK)
    out = _gather(idx, weight)
    return out.reshape(token_ids.shape + (DIM,))


# speedup vs baseline: 5.9046x; 1.7819x over previous
import functools

import jax
import jax.numpy as jnp
from jax import lax
from jax.experimental import pallas as pl
from jax.experimental.pallas import tpu as pltpu
from jax.experimental.pallas import tpu_sc as plsc

NUM_SEQS = 4096
SEQ = 50
DIM = 128
NC, NS = 2, 16
NW = NC * NS
SEQS_PER_W = NUM_SEQS // NW
RING = 8


def _make_kernel():
    mesh = plsc.VectorSubcoreMesh(core_axis_name="c", subcore_axis_name="s")

    @functools.partial(
        pl.kernel,
        out_type=jax.ShapeDtypeStruct((NUM_SEQS, SEQ, DIM), jnp.float32),
        mesh=mesh,
        scratch_types=[
            pltpu.VMEM((SEQS_PER_W, SEQ), jnp.int32),
            pltpu.VMEM((RING, SEQ, DIM), jnp.float32),
            pltpu.SemaphoreType.DMA((RING,)),
            pltpu.SemaphoreType.DMA((RING,)),
        ],
    )
    def gather_kernel(idx_hbm, table_hbm, out_hbm, idx_v, rows_v, sem_g, sem_o):
        wid = lax.axis_index("s") * NC + lax.axis_index("c")
        base = wid * SEQS_PER_W
        pltpu.sync_copy(idx_hbm.at[pl.ds(base, SEQS_PER_W)], idx_v)

        def gather(i, b):
            return pltpu.make_async_copy(
                table_hbm.at[idx_v.at[i]], rows_v.at[b], sem_g.at[b])

        def outcopy(i, b):
            return pltpu.make_async_copy(
                rows_v.at[b], out_hbm.at[base + i], sem_o.at[b])

        for b in range(RING):
            gather(b, b).start()

        @pl.loop(0, SEQS_PER_W - RING, step=RING)
        def _outer(t):
            for b in range(RING):
                gather(t + b, b).wait()
                outcopy(t + b, b).start()
            for b in range(RING):
                outcopy(t + b, b).wait()
                gather(t + RING + b, b).start()

        t = SEQS_PER_W - RING
        for b in range(RING):
            gather(t + b, b).wait()
            outcopy(t + b, b).start()
        for b in range(RING):
            outcopy(t + b, b).wait()

    return gather_kernel


_gather = _make_kernel()


def kernel(token_ids, weight):
    return _gather(token_ids.astype(jnp.int32), weight)


# speedup vs baseline: 5.9215x; 1.0029x over previous
import functools

import jax
import jax.numpy as jnp
from jax import lax
from jax.experimental import pallas as pl
from jax.experimental.pallas import tpu as pltpu
from jax.experimental.pallas import tpu_sc as plsc

NUM_SEQS = 4096
SEQ = 50
DIM = 128
NC, NS = 2, 16
NW = NC * NS
SEQS_PER_W = NUM_SEQS // NW
RING = 8


def _make_kernel():
    mesh = plsc.VectorSubcoreMesh(core_axis_name="c", subcore_axis_name="s")

    @functools.partial(
        pl.kernel,
        out_type=jax.ShapeDtypeStruct((NUM_SEQS, SEQ, DIM), jnp.float32),
        mesh=mesh,
        compiler_params=pltpu.CompilerParams(use_tc_tiling_on_sc=True),
        scratch_types=[
            pltpu.VMEM((SEQS_PER_W, SEQ), jnp.int32),
            pltpu.VMEM((RING, SEQ, DIM), jnp.float32),
            pltpu.SemaphoreType.DMA((RING,)),
            pltpu.SemaphoreType.DMA((RING,)),
        ],
    )
    def gather_kernel(idx_hbm, table_hbm, out_hbm, idx_v, rows_v, sem_g, sem_o):
        wid = lax.axis_index("s") * NC + lax.axis_index("c")
        base = wid * SEQS_PER_W
        pltpu.sync_copy(idx_hbm.at[pl.ds(base, SEQS_PER_W)], idx_v)

        def gather(i, b):
            return pltpu.make_async_copy(
                table_hbm.at[idx_v.at[i]], rows_v.at[b], sem_g.at[b])

        def outcopy(i, b):
            return pltpu.make_async_copy(
                rows_v.at[b], out_hbm.at[base + i], sem_o.at[b])

        for b in range(RING):
            gather(b, b).start()

        @pl.loop(0, SEQS_PER_W - RING, step=RING)
        def _outer(t):
            for b in range(RING):
                gather(t + b, b).wait()
                outcopy(t + b, b).start()
            for b in range(RING):
                outcopy(t + b, b).wait()
                gather(t + RING + b, b).start()

        t = SEQS_PER_W - RING
        for b in range(RING):
            gather(t + b, b).wait()
            outcopy(t + b, b).start()
        for b in range(RING):
            outcopy(t + b, b).wait()

    return gather_kernel


_gather = _make_kernel()


def kernel(token_ids, weight):
    return _gather(token_ids.astype(jnp.int32), weight)


# speedup vs baseline: 10.1708x; 1.7176x over previous
import functools

import jax
import jax.numpy as jnp
from jax import lax
from jax.experimental import pallas as pl
from jax.experimental.pallas import tpu as pltpu
from jax.experimental.pallas import tpu_sc as plsc

NUM_TOKENS = 4096 * 50
DIM = 128
NC, NS = 2, 16
NW = NC * NS
CHUNK = 128
CHUNKS_PER_W = NUM_TOKENS // (NW * CHUNK)
RING = 5


def _make_kernel():
    mesh = plsc.VectorSubcoreMesh(core_axis_name="c", subcore_axis_name="s")

    @functools.partial(
        pl.kernel,
        out_type=jax.ShapeDtypeStruct((NUM_TOKENS, DIM), jnp.float32),
        mesh=mesh,
        scratch_types=[
            pltpu.VMEM((NUM_TOKENS // NW,), jnp.int32),
            pltpu.VMEM((RING, CHUNK, DIM), jnp.float32),
            pltpu.SemaphoreType.DMA((RING,)),
            pltpu.SemaphoreType.DMA((RING,)),
        ],
    )
    def gather_kernel(idx_hbm, table_hbm, out_hbm, idx_v, rows_v, sem_g, sem_o):
        wid = lax.axis_index("s") * NC + lax.axis_index("c")
        base = wid * (CHUNKS_PER_W * CHUNK)
        pltpu.sync_copy(idx_hbm.at[pl.ds(base, CHUNKS_PER_W * CHUNK)], idx_v)

        def gather(j, b):
            return pltpu.make_async_copy(
                table_hbm.at[idx_v.at[pl.ds(j * CHUNK, CHUNK)]],
                rows_v.at[b], sem_g.at[b])

        def outcopy(j, b):
            return pltpu.make_async_copy(
                rows_v.at[b], out_hbm.at[pl.ds(base + j * CHUNK, CHUNK)],
                sem_o.at[b])

        for b in range(RING):
            gather(b, b).start()

        @pl.loop(0, CHUNKS_PER_W - RING, step=RING)
        def _outer(t):
            for b in range(RING):
                gather(t + b, b).wait()
                outcopy(t + b, b).start()
            for b in range(RING):
                outcopy(t + b, b).wait()
                gather(t + RING + b, b).start()

        t = CHUNKS_PER_W - RING
        for b in range(RING):
            gather(t + b, b).wait()
            outcopy(t + b, b).start()
        for b in range(RING):
            outcopy(t + b, b).wait()

    return gather_kernel


_gather = _make_kernel()


def kernel(token_ids, weight):
    idx = token_ids.astype(jnp.int32).T.reshape(NUM_TOKENS)
    out = _gather(idx, weight)
    seqs, seq_len = token_ids.shape
    return out.reshape(seq_len, seqs, DIM).transpose(1, 0, 2)


# speedup vs baseline: 10.4726x; 1.0297x over previous
import functools

import jax
import jax.numpy as jnp
from jax import lax
from jax.experimental import pallas as pl
from jax.experimental.pallas import tpu as pltpu
from jax.experimental.pallas import tpu_sc as plsc

NUM_TOKENS = 4096 * 50
DIM = 128
NC, NS = 2, 16
NW = NC * NS
CHUNK = 128
CHUNKS_PER_W = NUM_TOKENS // (NW * CHUNK)
RING = 7
DEPTH = 3


def _make_kernel():
    mesh = plsc.VectorSubcoreMesh(core_axis_name="c", subcore_axis_name="s")

    @functools.partial(
        pl.kernel,
        out_type=jax.ShapeDtypeStruct((NUM_TOKENS, DIM), jnp.float32),
        mesh=mesh,
        scratch_types=[
            pltpu.VMEM((NUM_TOKENS // NW,), jnp.int32),
            pltpu.VMEM((RING, CHUNK, DIM), jnp.float32),
            pltpu.SemaphoreType.DMA((RING,)),
            pltpu.SemaphoreType.DMA((RING,)),
        ],
    )
    def gather_kernel(idx_hbm, table_hbm, out_hbm, idx_v, rows_v, sem_g, sem_o):
        wid = lax.axis_index("s") * NC + lax.axis_index("c")
        base = wid * (CHUNKS_PER_W * CHUNK)
        pltpu.sync_copy(idx_hbm.at[pl.ds(base, CHUNKS_PER_W * CHUNK)], idx_v)

        def gather(j, b):
            return pltpu.make_async_copy(
                table_hbm.at[idx_v.at[pl.ds(j * CHUNK, CHUNK)]],
                rows_v.at[b], sem_g.at[b])

        def outcopy(j, b):
            return pltpu.make_async_copy(
                rows_v.at[b], out_hbm.at[pl.ds(base + j * CHUNK, CHUNK)],
                sem_o.at[b])

        for j in range(DEPTH):
            gather(j, j).start()

        @pl.loop(0, CHUNKS_PER_W)
        def _body(j):
            s = lax.rem(j, RING)
            gather(j, s).wait()
            outcopy(j, s).start()
            g = j + DEPTH

            @pl.when(g < CHUNKS_PER_W)
            def _():
                sg = lax.rem(g, RING)

                @pl.when(g >= RING)
                def _():
                    outcopy(g - RING, sg).wait()

                gather(g, sg).start()

        for d in range(RING):
            j = CHUNKS_PER_W - RING + d
            outcopy(j, j % RING).wait()

    return gather_kernel


_gather = _make_kernel()


def kernel(token_ids, weight):
    idx = token_ids.astype(jnp.int32).T.reshape(NUM_TOKENS)
    out = _gather(idx, weight)
    seqs, seq_len = token_ids.shape
    return out.reshape(seq_len, seqs, DIM).transpose(1, 0, 2)


# speedup vs baseline: 10.4896x; 1.0016x over previous
import functools

import jax
import jax.numpy as jnp
from jax import lax
from jax.experimental import pallas as pl
from jax.experimental.pallas import tpu as pltpu
from jax.experimental.pallas import tpu_sc as plsc

NUM_TOKENS = 4096 * 50
DIM = 128
NC, NS = 2, 16
NW = NC * NS
CHUNK = 128
CHUNKS_PER_W = NUM_TOKENS // (NW * CHUNK)
RING = 7
DEPTH = 5


def _make_kernel():
    mesh = plsc.VectorSubcoreMesh(core_axis_name="c", subcore_axis_name="s")

    @functools.partial(
        pl.kernel,
        out_type=jax.ShapeDtypeStruct((NUM_TOKENS, DIM), jnp.float32),
        mesh=mesh,
        scratch_types=[
            pltpu.VMEM((NUM_TOKENS // NW,), jnp.int32),
            pltpu.VMEM((RING, CHUNK, DIM), jnp.float32),
            pltpu.SemaphoreType.DMA((RING,)),
            pltpu.SemaphoreType.DMA((RING,)),
        ],
    )
    def gather_kernel(idx_hbm, table_hbm, out_hbm, idx_v, rows_v, sem_g, sem_o):
        wid = lax.axis_index("s") * NC + lax.axis_index("c")
        base = wid * (CHUNKS_PER_W * CHUNK)
        pltpu.sync_copy(idx_hbm.at[pl.ds(base, CHUNKS_PER_W * CHUNK)], idx_v)

        def gather(j, b):
            return pltpu.make_async_copy(
                table_hbm.at[idx_v.at[pl.ds(j * CHUNK, CHUNK)]],
                rows_v.at[b], sem_g.at[b])

        def outcopy(j, b):
            return pltpu.make_async_copy(
                rows_v.at[b], out_hbm.at[pl.ds(base + j * CHUNK, CHUNK)],
                sem_o.at[b])

        for j in range(DEPTH):
            gather(j, j).start()

        @pl.loop(0, CHUNKS_PER_W)
        def _body(j):
            s = lax.rem(j, RING)
            gather(j, s).wait()
            outcopy(j, s).start()
            g = j + DEPTH

            @pl.when(g < CHUNKS_PER_W)
            def _():
                sg = lax.rem(g, RING)

                @pl.when(g >= RING)
                def _():
                    outcopy(g - RING, sg).wait()

                gather(g, sg).start()

        for d in range(RING):
            j = CHUNKS_PER_W - RING + d
            outcopy(j, j % RING).wait()

    return gather_kernel


_gather = _make_kernel()


def kernel(token_ids, weight):
    idx = token_ids.astype(jnp.int32).T.reshape(NUM_TOKENS)
    out = _gather(idx, weight)
    seqs, seq_len = token_ids.shape
    return out.reshape(seq_len, seqs, DIM).transpose(1, 0, 2)
